# Initial kernel scaffold; baseline (speedup 1.0000x reference)
import jax
import jax.numpy as jnp
from jax.experimental import pallas as pl


def kernel(pred_probs, margins, edge_index, norm_weights, injection_mask, raw_weights, num_iter):
    raise NotImplementedError("write your pallas kernel here")


# speedup vs baseline: 31.7255x; 31.7255x over previous
import functools

import jax
import jax.numpy as jnp
from jax import lax
from jax.experimental import pallas as pl
from jax.experimental.pallas import tpu as pltpu
from jax.experimental.pallas import tpu_sc as plsc

ALPHA = 0.999
NC = 2
NS = 16
NW = NC * NS
ROWS = 128
SL = 8


@functools.lru_cache(maxsize=None)
def _make_sc_scatter(N_pad, C, G):
    NPS = N_pad // NS
    mesh = plsc.VectorSubcoreMesh(
        core_axis_name="c", subcore_axis_name="s", num_cores=NC, num_subcores=NS
    )

    @functools.partial(
        pl.kernel,
        out_type=jax.ShapeDtypeStruct((NC, N_pad, C), jnp.float32),
        mesh=mesh,
        compiler_params=pltpu.CompilerParams(use_tc_tiling_on_sc=False),
        scratch_types=[
            pltpu.VMEM_SHARED((N_pad, C), jnp.float32),
            pltpu.VMEM((SL, ROWS), jnp.int32),
            pltpu.VMEM((SL, ROWS), jnp.int32),
            pltpu.VMEM((SL, ROWS), jnp.float32),
            pltpu.VMEM((SL, ROWS, C), jnp.float32),
            pltpu.SemaphoreType.DMA,
            pltpu.SemaphoreType.DMA,
        ],
    )
    def sc_scatter(z_hbm, colb_hbm, rowb_hbm, wb_hbm, zeros_hbm, out_hbm,
                   acc_sh, colv, rowv, wv, rowsv, gsem, ssem):
        c = lax.axis_index("c")
        s = lax.axis_index("s")
        wid = s * NC + c

        pltpu.sync_copy(zeros_hbm, acc_sh.at[pl.ds(s * NPS, NPS)])
        plsc.subcore_barrier()

        def group(g, carry):
            base = (wid * G + g) * SL
            pltpu.sync_copy(colb_hbm.at[pl.ds(base, SL)], colv)
            pltpu.sync_copy(rowb_hbm.at[pl.ds(base, SL)], rowv)
            pltpu.sync_copy(wb_hbm.at[pl.ds(base, SL)], wv)
            gds = [
                pltpu.async_copy(z_hbm.at[colv.at[j]], rowsv.at[j], gsem)
                for j in range(SL)
            ]
            for d in gds:
                d.wait()
            for j in range(SL):
                @plsc.parallel_loop(0, ROWS, 16, unroll=2)
                def _scale(i):
                    wvec = wv[j, pl.ds(i, 16)]
                    for l in range(16):
                        rowsv[j, i + l, :] = rowsv[j, i + l, :] * wvec[l]
            sds = [
                pltpu.async_copy(rowsv.at[j], acc_sh.at[rowv.at[j]], ssem, add=True)
                for j in range(SL)
            ]
            for d in sds:
                d.wait()
            return carry

        lax.fori_loop(0, G, group, 0)
        plsc.subcore_barrier()
        pltpu.sync_copy(
            acc_sh.at[pl.ds(s * NPS, NPS)], out_hbm.at[c, pl.ds(s * NPS, NPS)]
        )

    return sc_scatter


def kernel(pred_probs, margins, edge_index, norm_weights, injection_mask,
           raw_weights, num_iter):
    N, C = pred_probs.shape
    E = edge_index.shape[1]

    conf = jnp.where(injection_mask, jax.nn.sigmoid(raw_weights), 0.0)
    preds = jnp.argmax(pred_probs, axis=1)
    onehot = jax.nn.one_hot(preds, C, dtype=jnp.float32)
    h = onehot * (conf * margins * injection_mask.astype(jnp.float32))[:, None]

    group_edges = NW * SL * ROWS
    G = -(-E // group_edges)
    E_pad = G * group_edges
    pad = E_pad - E
    row = jnp.pad(edge_index[0], (0, pad)).reshape(-1, ROWS)
    col = jnp.pad(edge_index[1], (0, pad)).reshape(-1, ROWS)
    wgt = jnp.pad(norm_weights * ALPHA, (0, pad)).reshape(-1, ROWS)

    NPS = -(-N // (NS * 8)) * 8
    N_pad = NPS * NS
    zeros_h = jnp.zeros((NPS, C), jnp.float32)

    scat = _make_sc_scatter(N_pad, C, G)
    h_scaled = (1.0 - ALPHA) * h

    def body(_, z):
        p = scat(z, col, row, wgt, zeros_h)
        return h_scaled + p[0, :N] + p[1, :N]

    return lax.fori_loop(0, num_iter, body, h)
